# Initial kernel scaffold; baseline (speedup 1.0000x reference)
import jax
import jax.numpy as jnp
from jax.experimental import pallas as pl


def kernel(x, edge_index, W1, b1, g1, be1, W2, b2, g2, be2):
    raise NotImplementedError("write your pallas kernel here")


# speedup vs baseline: 11.0401x; 11.0401x over previous
import functools

import jax
import jax.numpy as jnp
from jax import lax
from jax.experimental import pallas as pl
from jax.experimental.pallas import tpu as pltpu
from jax.experimental.pallas import tpu_sc as plsc

N = 10000
D = 128
E = 320000
NPAD = 10240
NC, NS = 2, 16
NW = NC * NS
CH = 128
EPT = 10112
EPAD = EPT * NW
NITER = EPT // CH
RPT = NPAD // NS
BN_EPS = 1e-5

@functools.lru_cache(maxsize=None)
def _mesh():
    return plsc.VectorSubcoreMesh(core_axis_name="c", subcore_axis_name="s")


def _deg_body(dst_hbm, out_hbm, zb, ones_b, idx_b, acc):
    c = lax.axis_index("c")
    s = lax.axis_index("s")
    zvec = jnp.zeros((16,), jnp.float32)
    evec = jnp.where(lax.iota(jnp.int32, 16) == 0, 1.0, 0.0)

    def initrow(i, carry):
        zb[i, :] = zvec
        ones_b[i, :] = evec
        return carry

    lax.fori_loop(0, CH, initrow, 0)
    for j in range(RPT // CH):
        pltpu.sync_copy(zb, acc.at[pl.ds(s * RPT + j * CH, CH)])
    plsc.subcore_barrier()

    base = (c * NS + s) * EPT

    def body(i, carry):
        pltpu.sync_copy(dst_hbm.at[pl.ds(base + i * CH, CH)], idx_b)
        pltpu.sync_copy(ones_b, acc.at[idx_b], add=True)
        return carry

    lax.fori_loop(0, NITER, body, 0)
    plsc.subcore_barrier()
    for j in range(RPT // CH):
        r0 = s * RPT + j * CH
        pltpu.sync_copy(acc.at[pl.ds(r0, CH)], out_hbm.at[c, pl.ds(r0, CH)])


@functools.lru_cache(maxsize=None)
def _deg_call():
    return pl.kernel(
        _deg_body,
        out_type=jax.ShapeDtypeStruct((NC, NPAD, 16), jnp.float32),
        mesh=_mesh(),
        scratch_types=[
            pltpu.VMEM((CH, 16), jnp.float32),
            pltpu.VMEM((CH, 16), jnp.float32),
            pltpu.VMEM((CH,), jnp.int32),
            pltpu.VMEM_SHARED((NPAD, 16), jnp.float32),
        ],
    )


def _scat_body(src_hbm, dst_hbm, p_hbm, out_hbm, zb, sidx, didx, rows, acc, sem):
    c = lax.axis_index("c")
    s = lax.axis_index("s")
    zvec = jnp.zeros((16,), jnp.float32)

    def zrow(i, carry):
        for k in range(D // 16):
            zb[i, pl.ds(k * 16, 16)] = zvec
        return carry

    lax.fori_loop(0, CH, zrow, 0)
    for j in range(RPT // CH):
        pltpu.sync_copy(zb, acc.at[pl.ds(s * RPT + j * CH, CH)])
    plsc.subcore_barrier()

    base = (c * NS + s) * EPT

    def body(i, carry):
        off = base + i * CH
        pltpu.sync_copy(src_hbm.at[pl.ds(off, CH)], sidx)
        pltpu.sync_copy(dst_hbm.at[pl.ds(off, CH)], didx)
        pltpu.async_copy(p_hbm.at[sidx], rows, sem).wait()
        pltpu.sync_copy(rows, acc.at[didx], add=True)
        return carry

    lax.fori_loop(0, NITER, body, 0)
    plsc.subcore_barrier()
    for j in range(RPT // CH):
        r0 = s * RPT + j * CH
        pltpu.sync_copy(acc.at[pl.ds(r0, CH)], out_hbm.at[c, pl.ds(r0, CH)])


@functools.lru_cache(maxsize=None)
def _scat_call():
    return pl.kernel(
        _scat_body,
        out_type=jax.ShapeDtypeStruct((NC, NPAD, D), jnp.float32),
        mesh=_mesh(),
        scratch_types=[
            pltpu.VMEM((CH, D), jnp.float32),
            pltpu.VMEM((CH,), jnp.int32),
            pltpu.VMEM((CH,), jnp.int32),
            pltpu.VMEM((CH, D), jnp.float32),
            pltpu.VMEM_SHARED((NPAD, D), jnp.float32),
            pltpu.SemaphoreType.DMA,
        ],
    )


def _mm_scale_body(degp_ref, x_ref, w_ref, p_ref, dinv_ref):
    dp = degp_ref[...]
    degsum = dp[0, :N, 0] + dp[1, :N, 0] + 1.0
    dinv = lax.rsqrt(degsum).reshape(N, 1)
    dinv_ref[...] = dinv
    p_ref[...] = (
        jnp.dot(x_ref[...], w_ref[...], preferred_element_type=jnp.float32) * dinv
    )


def _mid_body(s_ref, p_ref, dinv_ref, b_ref, g_ref, be_ref, w_ref, out_ref):
    sp = s_ref[...]
    dinv = dinv_ref[...]
    u = (sp[0, :N] + sp[1, :N] + p_ref[...]) * dinv + b_ref[...]
    mu = jnp.mean(u, axis=0)
    var = jnp.mean((u - mu) ** 2, axis=0)
    h = (u - mu) * lax.rsqrt(var + BN_EPS) * g_ref[...] + be_ref[...]
    h = jnp.maximum(h, 0.0)
    out_ref[...] = (
        jnp.dot(h, w_ref[...], preferred_element_type=jnp.float32) * dinv
    )


def _fin_body(s_ref, p_ref, dinv_ref, b_ref, g_ref, be_ref, out_ref):
    sp = s_ref[...]
    u = (sp[0, :N] + sp[1, :N] + p_ref[...]) * dinv_ref[...] + b_ref[...]
    mu = jnp.mean(u, axis=0)
    var = jnp.mean((u - mu) ** 2, axis=0)
    h = (u - mu) * lax.rsqrt(var + BN_EPS) * g_ref[...] + be_ref[...]
    out_ref[...] = jnp.maximum(h, 0.0)


def kernel(x, edge_index, W1, b1, g1, be1, W2, b2, g2, be2):
    src = edge_index[0].astype(jnp.int32)
    dst = edge_index[1].astype(jnp.int32)
    pad = EPAD - E
    src_p = jnp.concatenate([src, jnp.zeros((pad,), jnp.int32)])
    dst_p = jnp.concatenate([dst, jnp.full((pad,), N, jnp.int32)])

    degp = _deg_call()(dst_p)

    p1, dinv = pl.pallas_call(
        _mm_scale_body,
        out_shape=(
            jax.ShapeDtypeStruct((N, D), jnp.float32),
            jax.ShapeDtypeStruct((N, 1), jnp.float32),
        ),
    )(degp, x, W1)

    s1 = _scat_call()(src_p, dst_p, p1)

    p2 = pl.pallas_call(
        _mid_body,
        out_shape=jax.ShapeDtypeStruct((N, D), jnp.float32),
    )(s1, p1, dinv, b1, g1, be1, W2)

    s2 = _scat_call()(src_p, dst_p, p2)

    out = pl.pallas_call(
        _fin_body,
        out_shape=jax.ShapeDtypeStruct((N, D), jnp.float32),
    )(s2, p2, dinv, b2, g2, be2)

    return out
